# Initial kernel scaffold; baseline (speedup 1.0000x reference)
import jax
import jax.numpy as jnp
from jax.experimental import pallas as pl


def kernel(x, edge_index, batch, params):
    raise NotImplementedError("write your pallas kernel here")


# speedup vs baseline: 1.0668x; 1.0668x over previous
import functools

import jax
import jax.numpy as jnp
from jax import lax
from jax.experimental import pallas as pl
from jax.experimental.pallas import tpu as pltpu
from jax.experimental.pallas import tpu_sc as plsc

N = 10000
E = 320000
G = 256
H = 128

NC = 2
NS = 16
NW = NC * NS
ROWS = 320
N_PAD = NW * ROWS
SLICE = E // NW
CAP = 10240
CHUNK = 128

_mesh = plsc.VectorSubcoreMesh(
    core_axis_name="c", subcore_axis_name="s", num_cores=NC, num_subcores=NS)


def _wid():
    return lax.axis_index("s") * NC + lax.axis_index("c")


@functools.partial(
    pl.kernel,
    out_type=[
        jax.ShapeDtypeStruct((NW * NW * CAP,), jnp.int32),
        jax.ShapeDtypeStruct((NW * NW * CAP,), jnp.int32),
        jax.ShapeDtypeStruct((NW * NW,), jnp.int32),
    ],
    mesh=_mesh,
    compiler_params=pltpu.CompilerParams(needs_layout_passes=False),
    scratch_types=[
        pltpu.VMEM((SLICE,), jnp.int32),
        pltpu.VMEM((SLICE,), jnp.int32),
        pltpu.VMEM((CAP,), jnp.int32),
        pltpu.VMEM((CAP,), jnp.int32),
        pltpu.VMEM((NW,), jnp.int32),
    ],
)
def _bucketize(src_h, dst_h, bsrc_h, bdst_h, cnt_h,
               src_v, dst_v, csrc_v, cdst_v, cnt_v):
    p = _wid()
    pltpu.sync_copy(src_h.at[pl.ds(p * SLICE, SLICE)], src_v)
    pltpu.sync_copy(dst_h.at[pl.ds(p * SLICE, SLICE)], dst_v)

    zero16 = jnp.zeros((16,), jnp.int32)

    def _memset(i, _):
        csrc_v[pl.ds(i * 16, 16)] = zero16
        cdst_v[pl.ds(i * 16, 16)] = zero16
        return 0
    lax.fori_loop(0, CAP // 16, _memset, 0)
    cnt_v[pl.ds(0, 16)] = zero16
    cnt_v[pl.ds(16, 16)] = zero16

    lane = lax.iota(jnp.int32, 16)
    for t in range(NW):
        def _group(g, count, t=t):
            s16 = src_v[pl.ds(g * 16, 16)]
            d16 = dst_v[pl.ds(g * 16, 16)]
            own = (d16 * 6554) >> 21
            m = own == t
            mi = m.astype(jnp.int32)
            pos = count + plsc.cumsum(mi) - 1
            plsc.store_scatter(csrc_v, [pos], s16, mask=m)
            plsc.store_scatter(cdst_v, [pos], d16 - own * ROWS, mask=m)
            return count + jnp.sum(mi)
        cnt_t = lax.fori_loop(0, SLICE // 16, _group, jnp.int32(0))
        csrc_v[pl.ds(cnt_t, 16)] = zero16
        cdst_v[pl.ds(cnt_t, 16)] = jnp.full((16,), ROWS, jnp.int32)
        pltpu.sync_copy(csrc_v, bsrc_h.at[pl.ds((p * NW + t) * CAP, CAP)])
        pltpu.sync_copy(cdst_v, bdst_h.at[pl.ds((p * NW + t) * CAP, CAP)])
        half, slot = divmod(t, 16)
        old = cnt_v[pl.ds(half * 16, 16)]
        cnt_v[pl.ds(half * 16, 16)] = jnp.where(
            lane == slot, jnp.full((16,), cnt_t, jnp.int32), old)

    pltpu.sync_copy(cnt_v, cnt_h.at[pl.ds(p * NW, NW)])


@functools.partial(
    pl.kernel,
    out_type=jax.ShapeDtypeStruct((N_PAD * H,), jnp.float32),
    mesh=_mesh,
    compiler_params=pltpu.CompilerParams(needs_layout_passes=False),
    scratch_types=[
        pltpu.VMEM(((ROWS + 1) * H,), jnp.float32),
        pltpu.VMEM((CHUNK,), jnp.int32),
        pltpu.VMEM((CHUNK,), jnp.int32),
        pltpu.VMEM((CHUNK, H), jnp.float32),
        pltpu.VMEM((NW * NW,), jnp.int32),
        pltpu.SemaphoreType.DMA,
    ],
)
def _segmax(b_h, bsrc_h, bdst_h, cnt_h, out_h,
            acc_v, idx_v, ldst_v, rows_v, cnt_v, sem):
    t = _wid()
    neginf = jnp.full((16,), -jnp.inf, jnp.float32)

    def _init(i, _):
        acc_v[pl.ds(i * 16, 16)] = neginf
        return 0
    lax.fori_loop(0, (ROWS + 1) * H // 16, _init, 0)

    pltpu.sync_copy(cnt_h, cnt_v)
    t16 = jnp.full((16,), t, jnp.int32)

    def _bucket(p, _):
        q = plsc.load_gather(cnt_v, [jnp.full((16,), p * NW, jnp.int32) + t16])
        n = jnp.max(q)
        base = (p * NW + t) * CAP

        def _chunk(c, _, n=n, base=base):
            off = base + c * CHUNK
            pltpu.sync_copy(bsrc_h.at[pl.ds(off, CHUNK)], idx_v)
            pltpu.sync_copy(bdst_h.at[pl.ds(off, CHUNK)], ldst_v)
            pltpu.async_copy(b_h.at[idx_v], rows_v, sem).wait()
            m = jnp.minimum(CHUNK, n - c * CHUNK)

            def _grp(gg, _):
                d16 = ldst_v[pl.ds(gg * 16, 16)]
                for j in range(16):
                    d = d16[j]
                    for f in range(H // 16):
                        a = acc_v[pl.ds(d * H + f * 16, 16)]
                        r = rows_v[gg * 16 + j, pl.ds(f * 16, 16)]
                        acc_v[pl.ds(d * H + f * 16, 16)] = jnp.maximum(a, r)
                return 0
            lax.fori_loop(0, (m + 15) >> 4, _grp, 0)
            return 0
        lax.fori_loop(0, (n + CHUNK - 1) >> 7, _chunk, 0)
        return 0
    lax.fori_loop(0, NW, _bucket, 0)

    pltpu.sync_copy(acc_v.at[pl.ds(0, ROWS * H)],
                    out_h.at[pl.ds(t * ROWS * H, ROWS * H)])


def _tc_layer_body(combine, c_ref, m_ref, w1t_ref, w2t_ref, b_ref,
                   bv_ref, cout_ref):
    if combine:
        s = c_ref[...] + m_ref[...]
        h = jnp.where(jnp.isneginf(s), 0.0, s)
        h = jnp.maximum(h, 0.0)
    else:
        h = c_ref[...]
    bv = jnp.dot(h, w2t_ref[...], preferred_element_type=jnp.float32)
    cc = jnp.dot(h, w1t_ref[...], preferred_element_type=jnp.float32)
    bv_ref[...] = bv
    cout_ref[...] = cc + b_ref[...] - bv


_ROW_BLK = 1280


def _tc_layer(c_in, m_in, w1t, w2t, b, combine):
    grid = (N_PAD // _ROW_BLK,)
    big = pl.BlockSpec((_ROW_BLK, H), lambda i: (i, 0))
    wspec = pl.BlockSpec((H, H), lambda i: (0, 0))
    bspec = pl.BlockSpec((1, H), lambda i: (0, 0))
    return pl.pallas_call(
        functools.partial(_tc_layer_body, combine),
        grid=grid,
        in_specs=[big, big, wspec, wspec, bspec],
        out_specs=[big, big],
        out_shape=[jax.ShapeDtypeStruct((N_PAD, H), jnp.float32),
                   jax.ShapeDtypeStruct((N_PAD, H), jnp.float32)],
    )(c_in, m_in, w1t, w2t, b)


_POOL_BLK = 256
_NBLK = N_PAD // _POOL_BLK


def _heads_body(c_ref, m_ref, batch_ref,
                p1w, p1b, p2w, p2b, p3w, p3b,
                o1w, o1b, o2w, o2b, o3w, o3b,
                t1w, t1b, t2w, t2b,
                a1w, a1b, a2w, a2b, a3w, a3b,
                phys_ref, opt_ref, tox_ref, arom_ref,
                sums, cnts):
    i = pl.program_id(0)

    @pl.when(i == 0)
    def _():
        sums[...] = jnp.zeros_like(sums)
        cnts[...] = jnp.zeros_like(cnts)

    s = c_ref[...] + m_ref[...]
    h = jnp.where(jnp.isneginf(s), 0.0, s)
    bb = batch_ref[...].reshape(1, _POOL_BLK)
    g_iota = lax.broadcasted_iota(jnp.int32, (G, _POOL_BLK), 0)
    onehot = (bb == g_iota).astype(jnp.float32)
    sums[...] += jnp.dot(onehot, h, preferred_element_type=jnp.float32)
    cnts[...] += jnp.sum(onehot, axis=1, keepdims=True)

    @pl.when(i == _NBLK - 1)
    def _():
        g = sums[...] / jnp.maximum(cnts[...], 1.0)

        def lin(v, w_ref, b_ref):
            return jnp.dot(v, w_ref[...],
                           preferred_element_type=jnp.float32) + b_ref[...]

        r = jax.nn.relu
        phys = lin(r(lin(r(lin(g, p1w, p1b)), p2w, p2b)), p3w, p3b)
        opt = lin(r(lin(r(lin(g, o1w, o1b)), o2w, o2b)), o3w, o3b)
        tox = lin(r(lin(g, t1w, t1b)), t2w, t2b)
        arom = lin(r(lin(r(lin(g, a1w, a1b)), a2w, a2b)), a3w, a3b)
        phys_ref[...] = phys
        opt_ref[...] = opt
        tox_ref[...] = tox
        arom_ref[...] = arom


def _pool_heads(c_in, m_in, batch3d, hw):
    big = pl.BlockSpec((_POOL_BLK, H), lambda i: (i, 0))
    bspec = pl.BlockSpec((1, 1, _POOL_BLK), lambda i: (i, 0, 0))

    def wspec(a):
        return pl.BlockSpec(a.shape, lambda i: tuple(0 for _ in a.shape))

    in_specs = [big, big, bspec] + [wspec(a) for a in hw]
    out_specs = [pl.BlockSpec((G, d), lambda i: (0, 0)) for d in (4, 3, 2, 1)]
    out_shape = [jax.ShapeDtypeStruct((G, d), jnp.float32) for d in (4, 3, 2, 1)]
    return pl.pallas_call(
        _heads_body,
        grid=(_NBLK,),
        in_specs=in_specs,
        out_specs=out_specs,
        out_shape=out_shape,
        scratch_shapes=[pltpu.VMEM((G, H), jnp.float32),
                        pltpu.VMEM((G, 1), jnp.float32)],
    )(c_in, m_in, batch3d, *hw)


def _split_conv(wb, in_f):
    w, b = wb
    w1 = jnp.zeros((H, H), jnp.float32).at[:, :in_f].set(w[:, :in_f])
    w2 = jnp.zeros((H, H), jnp.float32).at[:, :in_f].set(w[:, in_f:])
    return w1.T, w2.T, b.reshape(1, H)


def kernel(x, edge_index, batch, params):
    src = edge_index[0].astype(jnp.int32)
    dst = edge_index[1].astype(jnp.int32)

    x_pad = jnp.zeros((N_PAD, H), jnp.float32).at[:N, :x.shape[1]].set(x)
    batch3d = jnp.full((N_PAD,), G, jnp.int32).at[:N].set(
        batch.astype(jnp.int32)).reshape(_NBLK, 1, _POOL_BLK)

    w1t_1, w2t_1, b_1 = _split_conv(params['conv1'], x.shape[1])
    w1t_2, w2t_2, b_2 = _split_conv(params['conv2'], H)
    w1t_3, w2t_3, b_3 = _split_conv(params['conv3'], H)

    bsrc, bdst, cnt = _bucketize(src, dst)

    bv1, c1 = _tc_layer(x_pad, x_pad, w1t_1, w2t_1, b_1, combine=False)
    m1 = _segmax(bv1.reshape(N_PAD, H), bsrc, bdst, cnt).reshape(N_PAD, H)
    bv2, c2 = _tc_layer(c1, m1, w1t_2, w2t_2, b_2, combine=True)
    m2 = _segmax(bv2, bsrc, bdst, cnt).reshape(N_PAD, H)
    bv3, c3 = _tc_layer(c2, m2, w1t_3, w2t_3, b_3, combine=True)
    m3 = _segmax(bv3, bsrc, bdst, cnt).reshape(N_PAD, H)

    hw = []
    for name, n_lay in (('phys', 3), ('opt', 3), ('tox', 2), ('oth', 3)):
        for j in range(1, n_lay + 1):
            w, b = params[f'{name}{j}']
            hw.append(w.T)
            hw.append(b.reshape(1, -1))

    return _pool_heads(c3, m3, batch3d, hw)
